# baseline (speedup 1.0000x reference)
import functools

import jax
import jax.numpy as jnp
from jax import lax
from jax.experimental import pallas as pl
from jax.experimental.pallas import tpu as pltpu
from jax.experimental.pallas import tpu_sc as plsc


def _make_gather(V, D, B, n_workers, nc):
    b_per_w = B // n_workers
    C = 800
    n_chunks = b_per_w // C
    mesh = plsc.VectorSubcoreMesh(core_axis_name="c", subcore_axis_name="s")

    @functools.partial(
        pl.kernel,
        mesh=mesh,
        compiler_params=pltpu.CompilerParams(use_tc_tiling_on_sc=False),
        out_type=jax.ShapeDtypeStruct((B, D), jnp.float32),
        scratch_types=[
            pltpu.VMEM((b_per_w,), jnp.int32),
            pltpu.VMEM((C, D), jnp.float32),
            pltpu.VMEM((C, D), jnp.float32),
            pltpu.SemaphoreType.DMA,
            pltpu.SemaphoreType.DMA,
        ],
    )
    def k(ids_hbm, table_hbm, out_hbm, idx_all, rows0, rows1, sem0, sem1):
        wid = lax.axis_index("s") * nc + lax.axis_index("c")
        base = wid * b_per_w
        pltpu.sync_copy(ids_hbm.at[pl.ds(base, b_per_w)], idx_all)
        bufs = (rows0, rows1)
        sems = (sem0, sem1)

        def start(i):
            return pltpu.async_copy(
                table_hbm.at[idx_all.at[pl.ds(i * C, C)]], bufs[i % 2], sems[i % 2]
            )

        cps = [None] * n_chunks
        cps[0] = start(0)
        for i in range(n_chunks):
            cps[i].wait()
            if i + 1 < n_chunks:
                cps[i + 1] = start(i + 1)
            pltpu.sync_copy(bufs[i % 2], out_hbm.at[pl.ds(base + i * C, C)])

    return k


def kernel(entity_ids, table):
    B0, S = entity_ids.shape
    V, D = table.shape
    H = B0 // 2
    B = H * S
    info = plsc.get_sparse_core_info()
    n_workers = info.num_cores * info.num_subcores
    g = _make_gather(V, D, B, n_workers, info.num_cores)
    ids = entity_ids.astype(jnp.int32)
    halves = []
    for h in range(2):
        ids_h = ids[h * H:(h + 1) * H].reshape(B)
        halves.append(g(ids_h, table).reshape(H, S, D))
    return jnp.concatenate(halves, axis=0)


# speedup vs baseline: 1.0661x; 1.0661x over previous
import functools

import jax
import jax.numpy as jnp
from jax import lax
from jax.experimental import pallas as pl
from jax.experimental.pallas import tpu as pltpu
from jax.experimental.pallas import tpu_sc as plsc


def _make_gather(V, D, B, n_workers, nc):
    b_per_w = B // n_workers
    C = 800
    n_chunks = b_per_w // C
    mesh = plsc.VectorSubcoreMesh(core_axis_name="c", subcore_axis_name="s")

    @functools.partial(
        pl.kernel,
        mesh=mesh,
        compiler_params=pltpu.CompilerParams(use_tc_tiling_on_sc=False),
        out_type=jax.ShapeDtypeStruct((B, D), jnp.float32),
        scratch_types=[
            pltpu.VMEM((b_per_w,), jnp.int32),
            pltpu.VMEM((C, D), jnp.float32),
            pltpu.VMEM((C, D), jnp.float32),
            pltpu.SemaphoreType.DMA,
            pltpu.SemaphoreType.DMA,
        ],
    )
    def k(ids_hbm, table_hbm, out_hbm, idx_all, rows0, rows1, sem0, sem1):
        wid = lax.axis_index("s") * nc + lax.axis_index("c")
        base = wid * b_per_w
        pltpu.sync_copy(ids_hbm.at[pl.ds(base, b_per_w)], idx_all)
        bufs = (rows0, rows1)
        sems = (sem0, sem1)

        def start(i):
            return pltpu.async_copy(
                table_hbm.at[idx_all.at[pl.ds(i * C, C)]], bufs[i % 2], sems[i % 2]
            )

        cps = [None] * n_chunks
        cps[0] = start(0)
        for i in range(n_chunks):
            cps[i].wait()
            if i + 1 < n_chunks:
                cps[i + 1] = start(i + 1)
            pltpu.sync_copy(bufs[i % 2], out_hbm.at[pl.ds(base + i * C, C)])

    return k


def kernel(entity_ids, table):
    B0, S = entity_ids.shape
    V, D = table.shape
    B = B0 * S
    info = plsc.get_sparse_core_info()
    n_workers = info.num_cores * info.num_subcores
    ids = entity_ids.reshape(B).astype(jnp.int32)
    out = _make_gather(V, D, B, n_workers, info.num_cores)(ids, table)
    return out.reshape(B0, S, D)
